# Initial kernel scaffold; baseline (speedup 1.0000x reference)
import jax
import jax.numpy as jnp
from jax.experimental import pallas as pl


def kernel(adj_rows, adj_cols, adj_vals, user_emb, item_emb, image_feats, text_feats, W_img, b_img, W_txt, b_txt, Wq1, bq1, Wq2, Wc1, bc1, Wc2):
    raise NotImplementedError("write your pallas kernel here")


# speedup vs baseline: 1.4495x; 1.4495x over previous
import functools

import jax
import jax.numpy as jnp
from jax import lax
from jax.experimental import pallas as pl
from jax.experimental.pallas import tpu as pltpu

_N_USERS = 16384
_N_ITEMS = 4096
_D = 64
_TOPK = 10
_LAMBDA = 0.5
_N_NODES = _N_USERS + _N_ITEMS
_BM = 256


def _simtopk_body(x_ref, xt_ref, vals_ref, idx_ref, dinv_ref):
    x = x_ref[...]
    xt = xt_ref[...]
    s = lax.dot_general(x, xt, (((1,), (1,)), ((), ())),
                        preferred_element_type=jnp.float32)
    iota = lax.broadcasted_iota(jnp.int32, (_BM, _N_ITEMS), 1)
    neg = jnp.float32(-jnp.inf)
    vs, js = [], []
    for _ in range(_TOPK):
        m = jnp.max(s, axis=1, keepdims=True)
        j = jnp.min(jnp.where(s >= m, iota, _N_ITEMS), axis=1, keepdims=True)
        vs.append(m)
        js.append(j)
        s = jnp.where(iota == j, neg, s)
    rowsum = vs[0]
    for v in vs[1:]:
        rowsum = rowsum + v
    dinv = jnp.where(rowsum > 0, lax.rsqrt(rowsum + 1e-8), 0.0)
    pad_f = jnp.zeros((_BM, 128 - _TOPK), jnp.float32)
    pad_i = jnp.zeros((_BM, 128 - _TOPK), jnp.int32)
    vals_ref[...] = jnp.concatenate(vs + [pad_f], axis=1)
    idx_ref[...] = jnp.concatenate(js + [pad_i], axis=1)
    dinv_ref[...] = jnp.broadcast_to(dinv, (_BM, 128))


def _simtopk(xn):
    f = xn.shape[1]
    vals, idx, dinv = pl.pallas_call(
        _simtopk_body,
        grid=(_N_ITEMS // _BM,),
        in_specs=[
            pl.BlockSpec((_BM, f), lambda i: (i, 0)),
            pl.BlockSpec((_N_ITEMS, f), lambda i: (0, 0)),
        ],
        out_specs=[
            pl.BlockSpec((_BM, 128), lambda i: (i, 0)),
            pl.BlockSpec((_BM, 128), lambda i: (i, 0)),
            pl.BlockSpec((_BM, 128), lambda i: (i, 0)),
        ],
        out_shape=[
            jax.ShapeDtypeStruct((_N_ITEMS, 128), jnp.float32),
            jax.ShapeDtypeStruct((_N_ITEMS, 128), jnp.int32),
            jax.ShapeDtypeStruct((_N_ITEMS, 128), jnp.float32),
        ],
    )(xn, xn)
    return vals[:, :_TOPK], idx[:, :_TOPK], dinv[:, 0]


def _normalize(x, eps):
    return x / (jnp.linalg.norm(x, axis=1, keepdims=True) + eps)


def _sparse_prop(vals, idx, dinv, emb):
    w = vals * dinv[:, None] * dinv[idx]
    return (w[..., None] * emb[idx]).sum(axis=1)


def kernel(adj_rows, adj_cols, adj_vals, user_emb, item_emb, image_feats,
           text_feats, W_img, b_img, W_txt, b_txt, Wq1, bq1, Wq2, Wc1, bc1,
           Wc2):
    image_f = image_feats @ W_img + b_img
    text_f = text_feats @ W_txt + b_txt

    xn_io = _normalize(image_feats, 1e-8)
    xn_to = _normalize(text_feats, 1e-8)
    xn_if = _normalize(image_f, 1e-8)
    xn_tf = _normalize(text_f, 1e-8)

    g_io = _simtopk(xn_io)
    g_to = _simtopk(xn_to)
    g_if = _simtopk(xn_if)
    g_tf = _simtopk(xn_tf)

    image_item = ((1.0 - _LAMBDA) * _sparse_prop(*g_if, item_emb)
                  + _LAMBDA * _sparse_prop(*g_io, item_emb))
    text_item = ((1.0 - _LAMBDA) * _sparse_prop(*g_tf, item_emb)
                 + _LAMBDA * _sparse_prop(*g_to, item_emb))

    q = lambda x, W1, b1, W2: jnp.tanh(x @ W1 + b1) @ W2
    att = jnp.concatenate(
        [q(image_item, Wq1, bq1, Wq2), q(text_item, Wq1, bq1, Wq2)], axis=-1)
    w = jax.nn.softmax(att, axis=-1)
    h = w[:, 0:1] * image_item + w[:, 1:2] * text_item

    ego = jnp.concatenate([user_emb, item_emb], axis=0)
    allemb = [ego]
    for _ in range(2):
        ego = jax.ops.segment_sum(adj_vals[:, None] * ego[adj_cols], adj_rows,
                                  num_segments=_N_NODES)
        allemb.append(ego)
    allemb = jnp.stack(allemb, axis=1).mean(axis=1)
    u_g, i_g = allemb[:_N_USERS], allemb[_N_USERS:]

    h_norm = h / (jnp.linalg.norm(h, axis=1, keepdims=True) + 1e-12)
    att2 = jnp.concatenate(
        [q(i_g, Wc1, bc1, Wc2), q(h_norm, Wc1, bc1, Wc2)], axis=-1)
    w2 = jax.nn.softmax(att2, axis=-1)
    i_g = w2[:, 0:1] * i_g + w2[:, 1:2] * h_norm
    return (u_g, i_g, image_item, text_item, h)


# speedup vs baseline: 6.3413x; 4.3748x over previous
import functools

import jax
import jax.numpy as jnp
from jax import lax
from jax.experimental import pallas as pl
from jax.experimental.pallas import tpu as pltpu
from jax.experimental.pallas import tpu_sc as plsc

_N_USERS = 16384
_N_ITEMS = 4096
_D = 64
_TOPK = 10
_LAMBDA = 0.5
_N_NODES = _N_USERS + _N_ITEMS
_BM = 256


def _simtopk_body(x_ref, xt_ref, vals_ref, idx_ref, dinv_ref):
    x = x_ref[...]
    xt = xt_ref[...]
    s = lax.dot_general(x, xt, (((1,), (1,)), ((), ())),
                        preferred_element_type=jnp.float32)
    iota = lax.broadcasted_iota(jnp.int32, (_BM, _N_ITEMS), 1)
    neg = jnp.float32(-jnp.inf)
    vs, js = [], []
    for _ in range(_TOPK):
        m = jnp.max(s, axis=1, keepdims=True)
        j = jnp.min(jnp.where(s >= m, iota, _N_ITEMS), axis=1, keepdims=True)
        vs.append(m)
        js.append(j)
        s = jnp.where(iota == j, neg, s)
    rowsum = vs[0]
    for v in vs[1:]:
        rowsum = rowsum + v
    dinv = jnp.where(rowsum > 0, lax.rsqrt(rowsum + 1e-8), 0.0)
    pad_f = jnp.zeros((_BM, 128 - _TOPK), jnp.float32)
    pad_i = jnp.zeros((_BM, 128 - _TOPK), jnp.int32)
    vals_ref[...] = jnp.concatenate(vs + [pad_f], axis=1)
    idx_ref[...] = jnp.concatenate(js + [pad_i], axis=1)
    dinv_ref[...] = jnp.broadcast_to(dinv, (_BM, 128))


def _simtopk(xn):
    f = xn.shape[1]
    vals, idx, dinv = pl.pallas_call(
        _simtopk_body,
        grid=(_N_ITEMS // _BM,),
        in_specs=[
            pl.BlockSpec((_BM, f), lambda i: (i, 0)),
            pl.BlockSpec((_N_ITEMS, f), lambda i: (0, 0)),
        ],
        out_specs=[
            pl.BlockSpec((_BM, 128), lambda i: (i, 0)),
            pl.BlockSpec((_BM, 128), lambda i: (i, 0)),
            pl.BlockSpec((_BM, 128), lambda i: (i, 0)),
        ],
        out_shape=[
            jax.ShapeDtypeStruct((_N_ITEMS, 128), jnp.float32),
            jax.ShapeDtypeStruct((_N_ITEMS, 128), jnp.int32),
            jax.ShapeDtypeStruct((_N_ITEMS, 128), jnp.float32),
        ],
    )(xn, xn)
    return vals[:, :_TOPK], idx[:, :_TOPK], dinv[:, 0]


_NNZ = 1 << 20
_NTILES = 32
_EPT = _NNZ // _NTILES
_CH = 128
_SUPER = 32
_NSUP = _EPT // (_CH * _SUPER)
_RPT = _N_NODES // 16


def _segsum_body(rows_h, cols_h, vals_h, ego_h, out_h,
                 acc, colsb, rowsb, valsb, gath, zbuf, gsem):
    c = lax.axis_index("c")
    s = lax.axis_index("s")
    wid = c * 16 + s
    chunk0 = wid * (_EPT // _CH)

    @pl.loop(0, _CH)
    def _zero(i):
        z = jnp.zeros((16,), jnp.float32)
        for d in range(4):
            zbuf[i, pl.ds(d * 16, 16)] = z

    @pl.loop(0, _RPT // _CH)
    def _zcp(t):
        pltpu.sync_copy(zbuf, acc.at[pl.ds(s * _RPT + t * _CH, _CH)])

    plsc.subcore_barrier()

    for sc in range(_NSUP):
        sbase = chunk0 + sc * _SUPER
        pltpu.sync_copy(rows_h.at[pl.ds(sbase, _SUPER)], rowsb)
        pltpu.sync_copy(cols_h.at[pl.ds(sbase, _SUPER)], colsb)
        pltpu.sync_copy(vals_h.at[pl.ds(sbase, _SUPER)], valsb)

        pltpu.async_copy(ego_h.at[colsb.at[0]], gath.at[0], gsem).wait()

        @pl.loop(0, _SUPER)
        def _chunk(k):
            b = lax.rem(k, 2)

            @pl.when(k + 1 < _SUPER)
            def _start():
                pltpu.async_copy(ego_h.at[colsb.at[k + 1]],
                                 gath.at[1 - b], gsem)

            @plsc.parallel_loop(0, _CH // 16, unroll=2)
            def _scale(e16):
                v16 = valsb[k, pl.ds(e16 * 16, 16)]
                for j in range(16):
                    w = v16[j]
                    e = e16 * 16 + j
                    for d in range(4):
                        sl = pl.ds(d * 16, 16)
                        gath[b, e, sl] = gath[b, e, sl] * w

            pltpu.sync_copy(gath.at[b], acc.at[rowsb.at[k]], add=True)

            @pl.when(k + 1 < _SUPER)
            def _drain():
                pltpu.make_async_copy(ego_h.at[colsb.at[k + 1]],
                                      gath.at[1 - b], gsem).wait()

    plsc.subcore_barrier()
    pltpu.sync_copy(acc.at[pl.ds(s * _RPT, _RPT)],
                    out_h.at[c].at[pl.ds(s * _RPT, _RPT)])


def _segsum_layer(rows2, cols2, vals2, ego):
    f = pl.kernel(
        _segsum_body,
        out_type=jax.ShapeDtypeStruct((2, _N_NODES, _D), jnp.float32),
        mesh=plsc.VectorSubcoreMesh(core_axis_name="c", subcore_axis_name="s"),
        scratch_types=[
            pltpu.VMEM_SHARED((_N_NODES, _D), jnp.float32),
            pltpu.VMEM((_SUPER, _CH), jnp.int32),
            pltpu.VMEM((_SUPER, _CH), jnp.int32),
            pltpu.VMEM((_SUPER, _CH), jnp.float32),
            pltpu.VMEM((2, _CH, _D), jnp.float32),
            pltpu.VMEM((_CH, _D), jnp.float32),
            pltpu.SemaphoreType.DMA,
        ],
        compiler_params=pltpu.CompilerParams(use_tc_tiling_on_sc=False),
    )
    return f(rows2, cols2, vals2, ego)


def _pairsum_body(p_ref, o_ref):
    o_ref[...] = p_ref[0] + p_ref[1]


def _pairsum(p):
    blk = 2048
    return pl.pallas_call(
        _pairsum_body,
        grid=(_N_NODES // blk,),
        in_specs=[pl.BlockSpec((2, blk, _D), lambda i: (0, i, 0))],
        out_specs=pl.BlockSpec((blk, _D), lambda i: (i, 0)),
        out_shape=jax.ShapeDtypeStruct((_N_NODES, _D), jnp.float32),
    )(p)


def _mean_body(e0_ref, e1_ref, p_ref, o_ref):
    o_ref[...] = (e0_ref[...] + e1_ref[...] + p_ref[0] + p_ref[1]) * (1.0 / 3.0)


def _mean3(ego0, ego1, p2):
    blk = 2048
    return pl.pallas_call(
        _mean_body,
        grid=(_N_NODES // blk,),
        in_specs=[
            pl.BlockSpec((blk, _D), lambda i: (i, 0)),
            pl.BlockSpec((blk, _D), lambda i: (i, 0)),
            pl.BlockSpec((2, blk, _D), lambda i: (0, i, 0)),
        ],
        out_specs=pl.BlockSpec((blk, _D), lambda i: (i, 0)),
        out_shape=jax.ShapeDtypeStruct((_N_NODES, _D), jnp.float32),
    )(ego0, ego1, p2)


def _normalize(x, eps):
    return x / (jnp.linalg.norm(x, axis=1, keepdims=True) + eps)


def _sparse_prop(vals, idx, dinv, emb):
    w = vals * dinv[:, None] * dinv[idx]
    return (w[..., None] * emb[idx]).sum(axis=1)


def kernel(adj_rows, adj_cols, adj_vals, user_emb, item_emb, image_feats,
           text_feats, W_img, b_img, W_txt, b_txt, Wq1, bq1, Wq2, Wc1, bc1,
           Wc2):
    image_f = image_feats @ W_img + b_img
    text_f = text_feats @ W_txt + b_txt

    xn_io = _normalize(image_feats, 1e-8)
    xn_to = _normalize(text_feats, 1e-8)
    xn_if = _normalize(image_f, 1e-8)
    xn_tf = _normalize(text_f, 1e-8)

    g_io = _simtopk(xn_io)
    g_to = _simtopk(xn_to)
    g_if = _simtopk(xn_if)
    g_tf = _simtopk(xn_tf)

    image_item = ((1.0 - _LAMBDA) * _sparse_prop(*g_if, item_emb)
                  + _LAMBDA * _sparse_prop(*g_io, item_emb))
    text_item = ((1.0 - _LAMBDA) * _sparse_prop(*g_tf, item_emb)
                 + _LAMBDA * _sparse_prop(*g_to, item_emb))

    q = lambda x, W1, b1, W2: jnp.tanh(x @ W1 + b1) @ W2
    att = jnp.concatenate(
        [q(image_item, Wq1, bq1, Wq2), q(text_item, Wq1, bq1, Wq2)], axis=-1)
    w = jax.nn.softmax(att, axis=-1)
    h = w[:, 0:1] * image_item + w[:, 1:2] * text_item

    ego0 = jnp.concatenate([user_emb, item_emb], axis=0)
    rows2 = adj_rows.reshape(_NNZ // _CH, _CH)
    cols2 = adj_cols.reshape(_NNZ // _CH, _CH)
    vals2 = adj_vals.reshape(_NNZ // _CH, _CH)
    p1 = _segsum_layer(rows2, cols2, vals2, ego0)
    ego1 = _pairsum(p1)
    p2 = _segsum_layer(rows2, cols2, vals2, ego1)
    allemb = _mean3(ego0, ego1, p2)
    u_g, i_g = allemb[:_N_USERS], allemb[_N_USERS:]

    h_norm = h / (jnp.linalg.norm(h, axis=1, keepdims=True) + 1e-12)
    att2 = jnp.concatenate(
        [q(i_g, Wc1, bc1, Wc2), q(h_norm, Wc1, bc1, Wc2)], axis=-1)
    w2 = jax.nn.softmax(att2, axis=-1)
    i_g = w2[:, 0:1] * i_g + w2[:, 1:2] * h_norm
    return (u_g, i_g, image_item, text_item, h)


# speedup vs baseline: 16.3018x; 2.5707x over previous
import functools

import jax
import jax.numpy as jnp
from jax import lax
from jax.experimental import pallas as pl
from jax.experimental.pallas import tpu as pltpu
from jax.experimental.pallas import tpu_sc as plsc

_N_USERS = 16384
_N_ITEMS = 4096
_D = 64
_TOPK = 10
_LAMBDA = 0.5
_N_NODES = _N_USERS + _N_ITEMS
_BM = 256


def _simtopk_body(x_ref, xt_ref, vals_ref, idx_ref, dinv_ref):
    x = x_ref[...]
    xt = xt_ref[...]
    s = lax.dot_general(x, xt, (((1,), (1,)), ((), ())),
                        preferred_element_type=jnp.float32)
    iota = lax.broadcasted_iota(jnp.int32, (_BM, _N_ITEMS), 1)
    neg = jnp.float32(-jnp.inf)
    vs, js = [], []
    for _ in range(_TOPK):
        m = jnp.max(s, axis=1, keepdims=True)
        j = jnp.min(jnp.where(s >= m, iota, _N_ITEMS), axis=1, keepdims=True)
        vs.append(m)
        js.append(j)
        s = jnp.where(iota == j, neg, s)
    rowsum = vs[0]
    for v in vs[1:]:
        rowsum = rowsum + v
    dinv = jnp.where(rowsum > 0, lax.rsqrt(rowsum + 1e-8), 0.0)
    pad_f = jnp.zeros((_BM, 128 - _TOPK), jnp.float32)
    pad_i = jnp.zeros((_BM, 128 - _TOPK), jnp.int32)
    vals_ref[...] = jnp.concatenate(vs + [pad_f], axis=1)
    idx_ref[...] = jnp.concatenate(js + [pad_i], axis=1)
    dinv_ref[...] = jnp.broadcast_to(dinv, (_BM, 128))


def _simtopk(xn):
    f = xn.shape[1]
    vals, idx, dinv = pl.pallas_call(
        _simtopk_body,
        grid=(_N_ITEMS // _BM,),
        in_specs=[
            pl.BlockSpec((_BM, f), lambda i: (i, 0)),
            pl.BlockSpec((_N_ITEMS, f), lambda i: (0, 0)),
        ],
        out_specs=[
            pl.BlockSpec((_BM, 128), lambda i: (i, 0)),
            pl.BlockSpec((_BM, 128), lambda i: (i, 0)),
            pl.BlockSpec((_BM, 128), lambda i: (i, 0)),
        ],
        out_shape=[
            jax.ShapeDtypeStruct((_N_ITEMS, 128), jnp.float32),
            jax.ShapeDtypeStruct((_N_ITEMS, 128), jnp.int32),
            jax.ShapeDtypeStruct((_N_ITEMS, 128), jnp.float32),
        ],
    )(xn, xn)
    return vals[:, :_TOPK], idx[:, :_TOPK], dinv[:, 0]


_NNZ = 1 << 20
_NTILES = 32
_EPT = _NNZ // _NTILES
_CH = 128
_SUPER = 32
_NSUP = _EPT // (_CH * _SUPER)
_RPT = _N_NODES // 16


def _segsum_body(rows_h, cols_h, vals_h, ego_h, out_h,
                 acc, colsb, rowsb, valsb, gath, zbuf, gsem):
    c = lax.axis_index("c")
    s = lax.axis_index("s")
    wid = c * 16 + s
    chunk0 = wid * (_EPT // _CH)

    @pl.loop(0, _CH)
    def _zero(i):
        z = jnp.zeros((16,), jnp.float32)
        for d in range(4):
            zbuf[i, pl.ds(d * 16, 16)] = z

    @pl.loop(0, _RPT // _CH)
    def _zcp(t):
        pltpu.sync_copy(zbuf, acc.at[pl.ds(s * _RPT + t * _CH, _CH)])

    plsc.subcore_barrier()

    for sc in range(_NSUP):
        sbase = chunk0 + sc * _SUPER
        pltpu.sync_copy(rows_h.at[pl.ds(sbase, _SUPER)], rowsb)
        pltpu.sync_copy(cols_h.at[pl.ds(sbase, _SUPER)], colsb)
        pltpu.sync_copy(vals_h.at[pl.ds(sbase, _SUPER)], valsb)

        pltpu.async_copy(ego_h.at[colsb.at[0]], gath.at[0], gsem).wait()

        @pl.loop(0, _SUPER)
        def _chunk(k):
            b = lax.rem(k, 2)

            @pl.when(k + 1 < _SUPER)
            def _start():
                pltpu.async_copy(ego_h.at[colsb.at[k + 1]],
                                 gath.at[1 - b], gsem)

            @plsc.parallel_loop(0, _CH // 16, unroll=2)
            def _scale(e16):
                v16 = valsb[k, pl.ds(e16 * 16, 16)]
                for j in range(16):
                    w = v16[j]
                    e = e16 * 16 + j
                    for d in range(4):
                        sl = pl.ds(d * 16, 16)
                        gath[b, e, sl] = gath[b, e, sl] * w

            pltpu.sync_copy(gath.at[b], acc.at[rowsb.at[k]], add=True)

            @pl.when(k + 1 < _SUPER)
            def _drain():
                pltpu.make_async_copy(ego_h.at[colsb.at[k + 1]],
                                      gath.at[1 - b], gsem).wait()

    plsc.subcore_barrier()
    pltpu.sync_copy(acc.at[pl.ds(s * _RPT, _RPT)],
                    out_h.at[c].at[pl.ds(s * _RPT, _RPT)])


def _segsum_layer(rows2, cols2, vals2, ego):
    f = pl.kernel(
        _segsum_body,
        out_type=jax.ShapeDtypeStruct((2, _N_NODES, _D), jnp.float32),
        mesh=plsc.VectorSubcoreMesh(core_axis_name="c", subcore_axis_name="s"),
        scratch_types=[
            pltpu.VMEM_SHARED((_N_NODES, _D), jnp.float32),
            pltpu.VMEM((_SUPER, _CH), jnp.int32),
            pltpu.VMEM((_SUPER, _CH), jnp.int32),
            pltpu.VMEM((_SUPER, _CH), jnp.float32),
            pltpu.VMEM((2, _CH, _D), jnp.float32),
            pltpu.VMEM((_CH, _D), jnp.float32),
            pltpu.SemaphoreType.DMA,
        ],
        compiler_params=pltpu.CompilerParams(use_tc_tiling_on_sc=False),
    )
    return f(rows2, cols2, vals2, ego)


_PROWS = _N_ITEMS // _NTILES
_PEDGE = _PROWS * _TOPK


def _prop_body(vf1_h, if1_h, d1_h, vf2_h, if2_h, d2_h,
               emb_h, out_h, idxs, idxsf, valssf, dtabf, gath, outs, gsem):
    c = lax.axis_index("c")
    s = lax.axis_index("s")
    wid = c * 16 + s

    for phase in range(2):
        valsf_h = (vf1_h, vf2_h)[phase]
        idxf_h = (if1_h, if2_h)[phase]
        dinv_h = (d1_h, d2_h)[phase]
        lam = (1.0 - _LAMBDA, _LAMBDA)[phase]

        for gr in range(10):
            pltpu.sync_copy(
                idxf_h.at[pl.ds(wid * _PEDGE + gr * _CH, _CH)], idxs.at[gr])
        for gr in range(10):
            pltpu.async_copy(emb_h.at[idxs.at[gr]],
                             gath.at[pl.ds(gr * _CH, _CH)], gsem)
        pltpu.sync_copy(idxf_h.at[pl.ds(wid * _PEDGE, _PEDGE)],
                        idxsf.at[pl.ds(0, _PEDGE)])
        pltpu.sync_copy(valsf_h.at[pl.ds(wid * _PEDGE, _PEDGE)],
                        valssf.at[pl.ds(0, _PEDGE)])
        pltpu.sync_copy(dinv_h, dtabf.at[pl.ds(0, _N_ITEMS)])

        for gr in range(10):
            pltpu.make_async_copy(emb_h.at[idxs.at[gr]],
                                  gath.at[pl.ds(gr * _CH, _CH)], gsem).wait()

        @plsc.parallel_loop(0, _PROWS, unroll=2)
        def _acc(r):
            div = dtabf[pl.ds(wid * _PROWS + r, 16)]
            lam_di = lam * div[0]
            idxrow = idxsf[pl.ds(r * _TOPK, 16)]
            valsrow = valssf[pl.ds(r * _TOPK, 16)]
            a = [jnp.zeros((16,), jnp.float32) for _ in range(4)]
            for k in range(_TOPK):
                djv = dtabf[pl.ds(idxrow[k], 16)]
                w = lam_di * valsrow[k] * djv[0]
                for d in range(4):
                    a[d] = a[d] + w * gath[r * _TOPK + k, pl.ds(d * 16, 16)]
            for d in range(4):
                sl = pl.ds(d * 16, 16)
                if phase == 0:
                    outs[r, sl] = a[d]
                else:
                    outs[r, sl] = outs[r, sl] + a[d]

    pltpu.sync_copy(outs, out_h.at[pl.ds(wid * _PROWS, _PROWS)])


def _knn_prop(g1, g2, emb):
    v1, i1, d1 = g1
    v2, i2, d2 = g2
    f = pl.kernel(
        _prop_body,
        out_type=jax.ShapeDtypeStruct((_N_ITEMS, _D), jnp.float32),
        mesh=plsc.VectorSubcoreMesh(core_axis_name="c", subcore_axis_name="s"),
        scratch_types=[
            pltpu.VMEM((10, _CH), jnp.int32),
            pltpu.VMEM((_PEDGE + 16,), jnp.int32),
            pltpu.VMEM((_PEDGE + 16,), jnp.float32),
            pltpu.VMEM((_N_ITEMS + 16,), jnp.float32),
            pltpu.VMEM((_PEDGE, _D), jnp.float32),
            pltpu.VMEM((_PROWS, _D), jnp.float32),
            pltpu.SemaphoreType.DMA,
        ],
        compiler_params=pltpu.CompilerParams(use_tc_tiling_on_sc=False),
    )
    flat = lambda a: a.reshape(_N_ITEMS * _TOPK)
    return f(flat(v1), flat(i1), d1, flat(v2), flat(i2), d2, emb)


def _pairsum_body(p_ref, o_ref):
    o_ref[...] = p_ref[0] + p_ref[1]


def _pairsum(p):
    blk = 2048
    return pl.pallas_call(
        _pairsum_body,
        grid=(_N_NODES // blk,),
        in_specs=[pl.BlockSpec((2, blk, _D), lambda i: (0, i, 0))],
        out_specs=pl.BlockSpec((blk, _D), lambda i: (i, 0)),
        out_shape=jax.ShapeDtypeStruct((_N_NODES, _D), jnp.float32),
    )(p)


def _mean_body(e0_ref, e1_ref, p_ref, o_ref):
    o_ref[...] = (e0_ref[...] + e1_ref[...] + p_ref[0] + p_ref[1]) * (1.0 / 3.0)


def _mean3(ego0, ego1, p2):
    blk = 2048
    return pl.pallas_call(
        _mean_body,
        grid=(_N_NODES // blk,),
        in_specs=[
            pl.BlockSpec((blk, _D), lambda i: (i, 0)),
            pl.BlockSpec((blk, _D), lambda i: (i, 0)),
            pl.BlockSpec((2, blk, _D), lambda i: (0, i, 0)),
        ],
        out_specs=pl.BlockSpec((blk, _D), lambda i: (i, 0)),
        out_shape=jax.ShapeDtypeStruct((_N_NODES, _D), jnp.float32),
    )(ego0, ego1, p2)


def _normalize(x, eps):
    return x / (jnp.linalg.norm(x, axis=1, keepdims=True) + eps)


def _sparse_prop(vals, idx, dinv, emb):
    w = vals * dinv[:, None] * dinv[idx]
    return (w[..., None] * emb[idx]).sum(axis=1)


def kernel(adj_rows, adj_cols, adj_vals, user_emb, item_emb, image_feats,
           text_feats, W_img, b_img, W_txt, b_txt, Wq1, bq1, Wq2, Wc1, bc1,
           Wc2):
    image_f = image_feats @ W_img + b_img
    text_f = text_feats @ W_txt + b_txt

    xn_io = _normalize(image_feats, 1e-8)
    xn_to = _normalize(text_feats, 1e-8)
    xn_if = _normalize(image_f, 1e-8)
    xn_tf = _normalize(text_f, 1e-8)

    g_io = _simtopk(xn_io)
    g_to = _simtopk(xn_to)
    g_if = _simtopk(xn_if)
    g_tf = _simtopk(xn_tf)

    image_item = _knn_prop(g_if, g_io, item_emb)
    text_item = _knn_prop(g_tf, g_to, item_emb)

    q = lambda x, W1, b1, W2: jnp.tanh(x @ W1 + b1) @ W2
    att = jnp.concatenate(
        [q(image_item, Wq1, bq1, Wq2), q(text_item, Wq1, bq1, Wq2)], axis=-1)
    w = jax.nn.softmax(att, axis=-1)
    h = w[:, 0:1] * image_item + w[:, 1:2] * text_item

    ego0 = jnp.concatenate([user_emb, item_emb], axis=0)
    rows2 = adj_rows.reshape(_NNZ // _CH, _CH)
    cols2 = adj_cols.reshape(_NNZ // _CH, _CH)
    vals2 = adj_vals.reshape(_NNZ // _CH, _CH)
    p1 = _segsum_layer(rows2, cols2, vals2, ego0)
    ego1 = _pairsum(p1)
    p2 = _segsum_layer(rows2, cols2, vals2, ego1)
    allemb = _mean3(ego0, ego1, p2)
    u_g, i_g = allemb[:_N_USERS], allemb[_N_USERS:]

    h_norm = h / (jnp.linalg.norm(h, axis=1, keepdims=True) + 1e-12)
    att2 = jnp.concatenate(
        [q(i_g, Wc1, bc1, Wc2), q(h_norm, Wc1, bc1, Wc2)], axis=-1)
    w2 = jax.nn.softmax(att2, axis=-1)
    i_g = w2[:, 0:1] * i_g + w2[:, 1:2] * h_norm
    return (u_g, i_g, image_item, text_item, h)


# speedup vs baseline: 16.8995x; 1.0367x over previous
import functools

import jax
import jax.numpy as jnp
from jax import lax
from jax.experimental import pallas as pl
from jax.experimental.pallas import tpu as pltpu
from jax.experimental.pallas import tpu_sc as plsc

_N_USERS = 16384
_N_ITEMS = 4096
_D = 64
_TOPK = 10
_LAMBDA = 0.5
_N_NODES = _N_USERS + _N_ITEMS
_BM = 256


def _simtopk_body(x_ref, xt_ref, vals_ref, idx_ref, dinv_ref):
    x = x_ref[...]
    xt = xt_ref[...]
    s = lax.dot_general(x, xt, (((1,), (1,)), ((), ())),
                        preferred_element_type=jnp.float32)
    iota = lax.broadcasted_iota(jnp.int32, (_BM, _N_ITEMS), 1)
    neg = jnp.float32(-jnp.inf)
    vs, js = [], []
    m = jnp.max(s, axis=1, keepdims=True)
    for k in range(_TOPK):
        j = jnp.min(jnp.where(s == m, iota, _N_ITEMS), axis=1, keepdims=True)
        vs.append(m)
        js.append(j)
        if k + 1 < _TOPK:
            m = jnp.max(jnp.where(s < m, s, neg), axis=1, keepdims=True)
    rowsum = vs[0]
    for v in vs[1:]:
        rowsum = rowsum + v
    dinv = jnp.where(rowsum > 0, lax.rsqrt(rowsum + 1e-8), 0.0)
    pad_f = jnp.zeros((_BM, 128 - _TOPK), jnp.float32)
    pad_i = jnp.zeros((_BM, 128 - _TOPK), jnp.int32)
    vals_ref[...] = jnp.concatenate(vs + [pad_f], axis=1)
    idx_ref[...] = jnp.concatenate(js + [pad_i], axis=1)
    dinv_ref[...] = jnp.broadcast_to(dinv, (_BM, 128))


def _simtopk(xn):
    f = xn.shape[1]
    vals, idx, dinv = pl.pallas_call(
        _simtopk_body,
        grid=(_N_ITEMS // _BM,),
        in_specs=[
            pl.BlockSpec((_BM, f), lambda i: (i, 0)),
            pl.BlockSpec((_N_ITEMS, f), lambda i: (0, 0)),
        ],
        out_specs=[
            pl.BlockSpec((_BM, 128), lambda i: (i, 0)),
            pl.BlockSpec((_BM, 128), lambda i: (i, 0)),
            pl.BlockSpec((_BM, 128), lambda i: (i, 0)),
        ],
        out_shape=[
            jax.ShapeDtypeStruct((_N_ITEMS, 128), jnp.float32),
            jax.ShapeDtypeStruct((_N_ITEMS, 128), jnp.int32),
            jax.ShapeDtypeStruct((_N_ITEMS, 128), jnp.float32),
        ],
    )(xn, xn)
    return vals[:, :_TOPK], idx[:, :_TOPK], dinv[:, 0]


_NNZ = 1 << 20
_NTILES = 32
_EPT = _NNZ // _NTILES
_CH = 128
_SUPER = 32
_NSUP = _EPT // (_CH * _SUPER)
_RPT = _N_NODES // 16


def _segsum_body(rows_h, cols_h, vals_h, ego_h, out_h,
                 acc, colsb, rowsb, valsb, gath, zbuf, gsem):
    c = lax.axis_index("c")
    s = lax.axis_index("s")
    wid = c * 16 + s
    chunk0 = wid * (_EPT // _CH)

    @pl.loop(0, _CH)
    def _zero(i):
        z = jnp.zeros((16,), jnp.float32)
        for d in range(4):
            zbuf[i, pl.ds(d * 16, 16)] = z

    @pl.loop(0, _RPT // _CH)
    def _zcp(t):
        pltpu.sync_copy(zbuf, acc.at[pl.ds(s * _RPT + t * _CH, _CH)])

    plsc.subcore_barrier()

    for sc in range(_NSUP):
        sbase = chunk0 + sc * _SUPER
        pltpu.sync_copy(rows_h.at[pl.ds(sbase, _SUPER)], rowsb)
        pltpu.sync_copy(cols_h.at[pl.ds(sbase, _SUPER)], colsb)
        pltpu.sync_copy(vals_h.at[pl.ds(sbase, _SUPER)], valsb)

        pltpu.async_copy(ego_h.at[colsb.at[0]], gath.at[0], gsem).wait()

        @pl.loop(0, _SUPER)
        def _chunk(k):
            b = lax.rem(k, 2)

            @pl.when(k + 1 < _SUPER)
            def _start():
                pltpu.async_copy(ego_h.at[colsb.at[k + 1]],
                                 gath.at[1 - b], gsem)

            @plsc.parallel_loop(0, _CH // 16, unroll=2)
            def _scale(e16):
                v16 = valsb[k, pl.ds(e16 * 16, 16)]
                for j in range(16):
                    w = v16[j]
                    e = e16 * 16 + j
                    for d in range(4):
                        sl = pl.ds(d * 16, 16)
                        gath[b, e, sl] = gath[b, e, sl] * w

            pltpu.sync_copy(gath.at[b], acc.at[rowsb.at[k]], add=True)

            @pl.when(k + 1 < _SUPER)
            def _drain():
                pltpu.make_async_copy(ego_h.at[colsb.at[k + 1]],
                                      gath.at[1 - b], gsem).wait()

    plsc.subcore_barrier()
    pltpu.sync_copy(acc.at[pl.ds(s * _RPT, _RPT)],
                    out_h.at[c].at[pl.ds(s * _RPT, _RPT)])


def _segsum_layer(rows2, cols2, vals2, ego):
    f = pl.kernel(
        _segsum_body,
        out_type=jax.ShapeDtypeStruct((2, _N_NODES, _D), jnp.float32),
        mesh=plsc.VectorSubcoreMesh(core_axis_name="c", subcore_axis_name="s"),
        scratch_types=[
            pltpu.VMEM_SHARED((_N_NODES, _D), jnp.float32),
            pltpu.VMEM((_SUPER, _CH), jnp.int32),
            pltpu.VMEM((_SUPER, _CH), jnp.int32),
            pltpu.VMEM((_SUPER, _CH), jnp.float32),
            pltpu.VMEM((2, _CH, _D), jnp.float32),
            pltpu.VMEM((_CH, _D), jnp.float32),
            pltpu.SemaphoreType.DMA,
        ],
        compiler_params=pltpu.CompilerParams(use_tc_tiling_on_sc=False),
    )
    return f(rows2, cols2, vals2, ego)


_PROWS = _N_ITEMS // _NTILES
_PEDGE = _PROWS * _TOPK


def _prop_body(vf1_h, if1_h, d1_h, vf2_h, if2_h, d2_h,
               emb_h, out_h, idxs, idxsf, valssf, dtabf, gath, outs, gsem):
    c = lax.axis_index("c")
    s = lax.axis_index("s")
    wid = c * 16 + s

    for phase in range(2):
        valsf_h = (vf1_h, vf2_h)[phase]
        idxf_h = (if1_h, if2_h)[phase]
        dinv_h = (d1_h, d2_h)[phase]
        lam = (1.0 - _LAMBDA, _LAMBDA)[phase]

        for gr in range(10):
            pltpu.sync_copy(
                idxf_h.at[pl.ds(wid * _PEDGE + gr * _CH, _CH)], idxs.at[gr])
        for gr in range(10):
            pltpu.async_copy(emb_h.at[idxs.at[gr]],
                             gath.at[pl.ds(gr * _CH, _CH)], gsem)
        pltpu.sync_copy(idxf_h.at[pl.ds(wid * _PEDGE, _PEDGE)],
                        idxsf.at[pl.ds(0, _PEDGE)])
        pltpu.sync_copy(valsf_h.at[pl.ds(wid * _PEDGE, _PEDGE)],
                        valssf.at[pl.ds(0, _PEDGE)])
        pltpu.sync_copy(dinv_h, dtabf.at[pl.ds(0, _N_ITEMS)])

        for gr in range(10):
            pltpu.make_async_copy(emb_h.at[idxs.at[gr]],
                                  gath.at[pl.ds(gr * _CH, _CH)], gsem).wait()

        @plsc.parallel_loop(0, _PROWS, unroll=2)
        def _acc(r):
            div = dtabf[pl.ds(wid * _PROWS + r, 16)]
            lam_di = lam * div[0]
            idxrow = idxsf[pl.ds(r * _TOPK, 16)]
            valsrow = valssf[pl.ds(r * _TOPK, 16)]
            a = [jnp.zeros((16,), jnp.float32) for _ in range(4)]
            for k in range(_TOPK):
                djv = dtabf[pl.ds(idxrow[k], 16)]
                w = lam_di * valsrow[k] * djv[0]
                for d in range(4):
                    a[d] = a[d] + w * gath[r * _TOPK + k, pl.ds(d * 16, 16)]
            for d in range(4):
                sl = pl.ds(d * 16, 16)
                if phase == 0:
                    outs[r, sl] = a[d]
                else:
                    outs[r, sl] = outs[r, sl] + a[d]

    pltpu.sync_copy(outs, out_h.at[pl.ds(wid * _PROWS, _PROWS)])


def _knn_prop(g1, g2, emb):
    v1, i1, d1 = g1
    v2, i2, d2 = g2
    f = pl.kernel(
        _prop_body,
        out_type=jax.ShapeDtypeStruct((_N_ITEMS, _D), jnp.float32),
        mesh=plsc.VectorSubcoreMesh(core_axis_name="c", subcore_axis_name="s"),
        scratch_types=[
            pltpu.VMEM((10, _CH), jnp.int32),
            pltpu.VMEM((_PEDGE + 16,), jnp.int32),
            pltpu.VMEM((_PEDGE + 16,), jnp.float32),
            pltpu.VMEM((_N_ITEMS + 16,), jnp.float32),
            pltpu.VMEM((_PEDGE, _D), jnp.float32),
            pltpu.VMEM((_PROWS, _D), jnp.float32),
            pltpu.SemaphoreType.DMA,
        ],
        compiler_params=pltpu.CompilerParams(use_tc_tiling_on_sc=False),
    )
    flat = lambda a: a.reshape(_N_ITEMS * _TOPK)
    return f(flat(v1), flat(i1), d1, flat(v2), flat(i2), d2, emb)


def _pairsum_body(p_ref, o_ref):
    o_ref[...] = p_ref[0] + p_ref[1]


def _pairsum(p):
    blk = 2048
    return pl.pallas_call(
        _pairsum_body,
        grid=(_N_NODES // blk,),
        in_specs=[pl.BlockSpec((2, blk, _D), lambda i: (0, i, 0))],
        out_specs=pl.BlockSpec((blk, _D), lambda i: (i, 0)),
        out_shape=jax.ShapeDtypeStruct((_N_NODES, _D), jnp.float32),
    )(p)


def _mean_body(e0_ref, e1_ref, p_ref, o_ref):
    o_ref[...] = (e0_ref[...] + e1_ref[...] + p_ref[0] + p_ref[1]) * (1.0 / 3.0)


def _mean3(ego0, ego1, p2):
    blk = 2048
    return pl.pallas_call(
        _mean_body,
        grid=(_N_NODES // blk,),
        in_specs=[
            pl.BlockSpec((blk, _D), lambda i: (i, 0)),
            pl.BlockSpec((blk, _D), lambda i: (i, 0)),
            pl.BlockSpec((2, blk, _D), lambda i: (0, i, 0)),
        ],
        out_specs=pl.BlockSpec((blk, _D), lambda i: (i, 0)),
        out_shape=jax.ShapeDtypeStruct((_N_NODES, _D), jnp.float32),
    )(ego0, ego1, p2)


def _normalize(x, eps):
    return x / (jnp.linalg.norm(x, axis=1, keepdims=True) + eps)


def _sparse_prop(vals, idx, dinv, emb):
    w = vals * dinv[:, None] * dinv[idx]
    return (w[..., None] * emb[idx]).sum(axis=1)


def kernel(adj_rows, adj_cols, adj_vals, user_emb, item_emb, image_feats,
           text_feats, W_img, b_img, W_txt, b_txt, Wq1, bq1, Wq2, Wc1, bc1,
           Wc2):
    image_f = image_feats @ W_img + b_img
    text_f = text_feats @ W_txt + b_txt

    xn_io = _normalize(image_feats, 1e-8)
    xn_to = _normalize(text_feats, 1e-8)
    xn_if = _normalize(image_f, 1e-8)
    xn_tf = _normalize(text_f, 1e-8)

    g_io = _simtopk(xn_io)
    g_to = _simtopk(xn_to)
    g_if = _simtopk(xn_if)
    g_tf = _simtopk(xn_tf)

    image_item = _knn_prop(g_if, g_io, item_emb)
    text_item = _knn_prop(g_tf, g_to, item_emb)

    q = lambda x, W1, b1, W2: jnp.tanh(x @ W1 + b1) @ W2
    att = jnp.concatenate(
        [q(image_item, Wq1, bq1, Wq2), q(text_item, Wq1, bq1, Wq2)], axis=-1)
    w = jax.nn.softmax(att, axis=-1)
    h = w[:, 0:1] * image_item + w[:, 1:2] * text_item

    ego0 = jnp.concatenate([user_emb, item_emb], axis=0)
    rows2 = adj_rows.reshape(_NNZ // _CH, _CH)
    cols2 = adj_cols.reshape(_NNZ // _CH, _CH)
    vals2 = adj_vals.reshape(_NNZ // _CH, _CH)
    p1 = _segsum_layer(rows2, cols2, vals2, ego0)
    ego1 = _pairsum(p1)
    p2 = _segsum_layer(rows2, cols2, vals2, ego1)
    allemb = _mean3(ego0, ego1, p2)
    u_g, i_g = allemb[:_N_USERS], allemb[_N_USERS:]

    h_norm = h / (jnp.linalg.norm(h, axis=1, keepdims=True) + 1e-12)
    att2 = jnp.concatenate(
        [q(i_g, Wc1, bc1, Wc2), q(h_norm, Wc1, bc1, Wc2)], axis=-1)
    w2 = jax.nn.softmax(att2, axis=-1)
    i_g = w2[:, 0:1] * i_g + w2[:, 1:2] * h_norm
    return (u_g, i_g, image_item, text_item, h)


# speedup vs baseline: 18.3933x; 1.0884x over previous
import functools

import jax
import jax.numpy as jnp
from jax import lax
from jax.experimental import pallas as pl
from jax.experimental.pallas import tpu as pltpu
from jax.experimental.pallas import tpu_sc as plsc

_N_USERS = 16384
_N_ITEMS = 4096
_D = 64
_TOPK = 10
_LAMBDA = 0.5
_N_NODES = _N_USERS + _N_ITEMS
_BM = 256


def _simtopk_body(x_ref, xt_ref, vals_ref, idx_ref, dinv_ref):
    x = x_ref[...]
    xt = xt_ref[...]
    s = lax.dot_general(x, xt, (((1,), (1,)), ((), ())),
                        preferred_element_type=jnp.float32)
    iota = lax.broadcasted_iota(jnp.int32, (_BM, _N_ITEMS), 1).astype(
        jnp.float32)
    neg = jnp.float32(-jnp.inf)
    vs, js = [], []
    m = jnp.max(s, axis=1, keepdims=True)
    for k in range(_TOPK):
        j = jnp.sum(jnp.where(s == m, iota, 0.0), axis=1, keepdims=True)
        vs.append(m)
        js.append(j.astype(jnp.int32))
        if k + 1 < _TOPK:
            m = jnp.max(jnp.where(s < m, s, neg), axis=1, keepdims=True)
    rowsum = vs[0]
    for v in vs[1:]:
        rowsum = rowsum + v
    dinv = jnp.where(rowsum > 0, lax.rsqrt(rowsum + 1e-8), 0.0)
    pad_f = jnp.zeros((_BM, 128 - _TOPK), jnp.float32)
    pad_i = jnp.zeros((_BM, 128 - _TOPK), jnp.int32)
    vals_ref[...] = jnp.concatenate(vs + [pad_f], axis=1)
    idx_ref[...] = jnp.concatenate(js + [pad_i], axis=1)
    dinv_ref[...] = jnp.broadcast_to(dinv, (_BM, 128))


def _simtopk(xn):
    f = xn.shape[1]
    vals, idx, dinv = pl.pallas_call(
        _simtopk_body,
        grid=(_N_ITEMS // _BM,),
        in_specs=[
            pl.BlockSpec((_BM, f), lambda i: (i, 0)),
            pl.BlockSpec((_N_ITEMS, f), lambda i: (0, 0)),
        ],
        out_specs=[
            pl.BlockSpec((_BM, 128), lambda i: (i, 0)),
            pl.BlockSpec((_BM, 128), lambda i: (i, 0)),
            pl.BlockSpec((_BM, 128), lambda i: (i, 0)),
        ],
        out_shape=[
            jax.ShapeDtypeStruct((_N_ITEMS, 128), jnp.float32),
            jax.ShapeDtypeStruct((_N_ITEMS, 128), jnp.int32),
            jax.ShapeDtypeStruct((_N_ITEMS, 128), jnp.float32),
        ],
    )(xn, xn)
    return vals[:, :_TOPK], idx[:, :_TOPK], dinv[:, 0]


_NNZ = 1 << 20
_NTILES = 32
_EPT = _NNZ // _NTILES
_CH = 128
_SUPER = 32
_NSUP = _EPT // (_CH * _SUPER)
_RPT = _N_NODES // 16


def _segsum_body(rows_h, cols_h, vals_h, ego_h, out_h,
                 acc, colsb, rowsb, valsb, gath, zbuf, gsem):
    c = lax.axis_index("c")
    s = lax.axis_index("s")
    wid = c * 16 + s
    chunk0 = wid * (_EPT // _CH)

    @pl.loop(0, _CH)
    def _zero(i):
        z = jnp.zeros((16,), jnp.float32)
        for d in range(4):
            zbuf[i, pl.ds(d * 16, 16)] = z

    @pl.loop(0, _RPT // _CH)
    def _zcp(t):
        pltpu.sync_copy(zbuf, acc.at[pl.ds(s * _RPT + t * _CH, _CH)])

    plsc.subcore_barrier()

    for sc in range(_NSUP):
        sbase = chunk0 + sc * _SUPER
        pltpu.sync_copy(rows_h.at[pl.ds(sbase, _SUPER)], rowsb)
        pltpu.sync_copy(cols_h.at[pl.ds(sbase, _SUPER)], colsb)
        pltpu.sync_copy(vals_h.at[pl.ds(sbase, _SUPER)], valsb)

        pltpu.async_copy(ego_h.at[colsb.at[0]], gath.at[0], gsem).wait()

        @pl.loop(0, _SUPER)
        def _chunk(k):
            b = lax.rem(k, 2)

            @pl.when(k + 1 < _SUPER)
            def _start():
                pltpu.async_copy(ego_h.at[colsb.at[k + 1]],
                                 gath.at[1 - b], gsem)

            @plsc.parallel_loop(0, _CH // 16, unroll=2)
            def _scale(e16):
                v16 = valsb[k, pl.ds(e16 * 16, 16)]
                for j in range(16):
                    w = v16[j]
                    e = e16 * 16 + j
                    for d in range(4):
                        sl = pl.ds(d * 16, 16)
                        gath[b, e, sl] = gath[b, e, sl] * w

            pltpu.sync_copy(gath.at[b], acc.at[rowsb.at[k]], add=True)

            @pl.when(k + 1 < _SUPER)
            def _drain():
                pltpu.make_async_copy(ego_h.at[colsb.at[k + 1]],
                                      gath.at[1 - b], gsem).wait()

    plsc.subcore_barrier()
    pltpu.sync_copy(acc.at[pl.ds(s * _RPT, _RPT)],
                    out_h.at[c].at[pl.ds(s * _RPT, _RPT)])


def _segsum_layer(rows2, cols2, vals2, ego):
    f = pl.kernel(
        _segsum_body,
        out_type=jax.ShapeDtypeStruct((2, _N_NODES, _D), jnp.float32),
        mesh=plsc.VectorSubcoreMesh(core_axis_name="c", subcore_axis_name="s"),
        scratch_types=[
            pltpu.VMEM_SHARED((_N_NODES, _D), jnp.float32),
            pltpu.VMEM((_SUPER, _CH), jnp.int32),
            pltpu.VMEM((_SUPER, _CH), jnp.int32),
            pltpu.VMEM((_SUPER, _CH), jnp.float32),
            pltpu.VMEM((2, _CH, _D), jnp.float32),
            pltpu.VMEM((_CH, _D), jnp.float32),
            pltpu.SemaphoreType.DMA,
        ],
        compiler_params=pltpu.CompilerParams(use_tc_tiling_on_sc=False),
    )
    return f(rows2, cols2, vals2, ego)


_PROWS = _N_ITEMS // _NTILES
_PEDGE = _PROWS * _TOPK


def _prop_body(vf1_h, if1_h, d1_h, vf2_h, if2_h, d2_h,
               emb_h, out_h, idxs, idxsf, valssf, dtabf, gath, outs, gsem):
    c = lax.axis_index("c")
    s = lax.axis_index("s")
    wid = c * 16 + s

    for phase in range(2):
        valsf_h = (vf1_h, vf2_h)[phase]
        idxf_h = (if1_h, if2_h)[phase]
        dinv_h = (d1_h, d2_h)[phase]
        lam = (1.0 - _LAMBDA, _LAMBDA)[phase]

        for gr in range(10):
            pltpu.sync_copy(
                idxf_h.at[pl.ds(wid * _PEDGE + gr * _CH, _CH)], idxs.at[gr])
        for gr in range(10):
            pltpu.async_copy(emb_h.at[idxs.at[gr]],
                             gath.at[pl.ds(gr * _CH, _CH)], gsem)
        pltpu.sync_copy(idxf_h.at[pl.ds(wid * _PEDGE, _PEDGE)],
                        idxsf.at[pl.ds(0, _PEDGE)])
        pltpu.sync_copy(valsf_h.at[pl.ds(wid * _PEDGE, _PEDGE)],
                        valssf.at[pl.ds(0, _PEDGE)])
        pltpu.sync_copy(dinv_h, dtabf.at[pl.ds(0, _N_ITEMS)])

        for gr in range(10):
            pltpu.make_async_copy(emb_h.at[idxs.at[gr]],
                                  gath.at[pl.ds(gr * _CH, _CH)], gsem).wait()

        @plsc.parallel_loop(0, _PROWS, unroll=2)
        def _acc(r):
            div = dtabf[pl.ds(wid * _PROWS + r, 16)]
            lam_di = lam * div[0]
            idxrow = idxsf[pl.ds(r * _TOPK, 16)]
            valsrow = valssf[pl.ds(r * _TOPK, 16)]
            a = [jnp.zeros((16,), jnp.float32) for _ in range(4)]
            for k in range(_TOPK):
                djv = dtabf[pl.ds(idxrow[k], 16)]
                w = lam_di * valsrow[k] * djv[0]
                for d in range(4):
                    a[d] = a[d] + w * gath[r * _TOPK + k, pl.ds(d * 16, 16)]
            for d in range(4):
                sl = pl.ds(d * 16, 16)
                if phase == 0:
                    outs[r, sl] = a[d]
                else:
                    outs[r, sl] = outs[r, sl] + a[d]

    pltpu.sync_copy(outs, out_h.at[pl.ds(wid * _PROWS, _PROWS)])


def _knn_prop(g1, g2, emb):
    v1, i1, d1 = g1
    v2, i2, d2 = g2
    f = pl.kernel(
        _prop_body,
        out_type=jax.ShapeDtypeStruct((_N_ITEMS, _D), jnp.float32),
        mesh=plsc.VectorSubcoreMesh(core_axis_name="c", subcore_axis_name="s"),
        scratch_types=[
            pltpu.VMEM((10, _CH), jnp.int32),
            pltpu.VMEM((_PEDGE + 16,), jnp.int32),
            pltpu.VMEM((_PEDGE + 16,), jnp.float32),
            pltpu.VMEM((_N_ITEMS + 16,), jnp.float32),
            pltpu.VMEM((_PEDGE, _D), jnp.float32),
            pltpu.VMEM((_PROWS, _D), jnp.float32),
            pltpu.SemaphoreType.DMA,
        ],
        compiler_params=pltpu.CompilerParams(use_tc_tiling_on_sc=False),
    )
    flat = lambda a: a.reshape(_N_ITEMS * _TOPK)
    return f(flat(v1), flat(i1), d1, flat(v2), flat(i2), d2, emb)


def _pairsum_body(p_ref, o_ref):
    o_ref[...] = p_ref[0] + p_ref[1]


def _pairsum(p):
    blk = 2048
    return pl.pallas_call(
        _pairsum_body,
        grid=(_N_NODES // blk,),
        in_specs=[pl.BlockSpec((2, blk, _D), lambda i: (0, i, 0))],
        out_specs=pl.BlockSpec((blk, _D), lambda i: (i, 0)),
        out_shape=jax.ShapeDtypeStruct((_N_NODES, _D), jnp.float32),
    )(p)


def _mean_body(e0_ref, e1_ref, p_ref, o_ref):
    o_ref[...] = (e0_ref[...] + e1_ref[...] + p_ref[0] + p_ref[1]) * (1.0 / 3.0)


def _mean3(ego0, ego1, p2):
    blk = 2048
    return pl.pallas_call(
        _mean_body,
        grid=(_N_NODES // blk,),
        in_specs=[
            pl.BlockSpec((blk, _D), lambda i: (i, 0)),
            pl.BlockSpec((blk, _D), lambda i: (i, 0)),
            pl.BlockSpec((2, blk, _D), lambda i: (0, i, 0)),
        ],
        out_specs=pl.BlockSpec((blk, _D), lambda i: (i, 0)),
        out_shape=jax.ShapeDtypeStruct((_N_NODES, _D), jnp.float32),
    )(ego0, ego1, p2)


def _normalize(x, eps):
    return x / (jnp.linalg.norm(x, axis=1, keepdims=True) + eps)


def _featnorm_body(x_ref, w_ref, b_ref, o_ref, *, transform):
    x = x_ref[...]
    if transform:
        x = lax.dot_general(x, w_ref[...], (((1,), (0,)), ((), ())),
                            preferred_element_type=jnp.float32) + b_ref[...]
    n = jnp.sqrt(jnp.sum(x * x, axis=1, keepdims=True))
    o_ref[...] = x / (n + 1e-8)


def _featnorm(x, w=None, b=None):
    transform = w is not None
    blk = 1024
    f_in = x.shape[1]
    f_out = w.shape[1] if transform else f_in
    body = functools.partial(_featnorm_body, transform=transform)
    if not transform:
        w = jnp.zeros((8, 128), jnp.float32)
        b = jnp.zeros((1, 128), jnp.float32)
    else:
        b = b.reshape(1, f_out)
    return pl.pallas_call(
        body,
        grid=(_N_ITEMS // blk,),
        in_specs=[
            pl.BlockSpec((blk, f_in), lambda i: (i, 0)),
            pl.BlockSpec(w.shape, lambda i: (0, 0)),
            pl.BlockSpec(b.shape, lambda i: (0, 0)),
        ],
        out_specs=pl.BlockSpec((blk, f_out), lambda i: (i, 0)),
        out_shape=jax.ShapeDtypeStruct((_N_ITEMS, f_out), jnp.float32),
    )(x, w, b)


def _attn_body(a_ref, b_ref, w1_ref, b1_ref, w2_ref, o_ref, *, norm_b):
    a = a_ref[...]
    bb = b_ref[...]
    if norm_b:
        n = jnp.sqrt(jnp.sum(bb * bb, axis=1, keepdims=True))
        bb = bb / (n + 1e-12)
    w1 = w1_ref[...]
    b1 = b1_ref[...]
    w2 = w2_ref[...]
    q = lambda x: lax.dot_general(
        jnp.tanh(lax.dot_general(x, w1, (((1,), (0,)), ((), ())),
                                 preferred_element_type=jnp.float32) + b1),
        w2, (((1,), (0,)), ((), ())), preferred_element_type=jnp.float32)
    qa = q(a)[:, 0:1]
    qb = q(bb)[:, 0:1]
    wa = 1.0 / (1.0 + jnp.exp(qb - qa))
    o_ref[...] = wa * a + (1.0 - wa) * bb


def _attn_fuse(a, b, w1, b1, w2, norm_b=False):
    body = functools.partial(_attn_body, norm_b=norm_b)
    w2p = jnp.pad(w2, ((0, 0), (0, 127)))
    return pl.pallas_call(
        body,
        grid=(2,),
        in_specs=[
            pl.BlockSpec((_N_ITEMS // 2, _D), lambda i: (i, 0)),
            pl.BlockSpec((_N_ITEMS // 2, _D), lambda i: (i, 0)),
            pl.BlockSpec((_D, _D), lambda i: (0, 0)),
            pl.BlockSpec((1, _D), lambda i: (0, 0)),
            pl.BlockSpec((_D, 128), lambda i: (0, 0)),
        ],
        out_specs=pl.BlockSpec((_N_ITEMS // 2, _D), lambda i: (i, 0)),
        out_shape=jax.ShapeDtypeStruct((_N_ITEMS, _D), jnp.float32),
    )(a, b, w1, b1, w2p)


def kernel(adj_rows, adj_cols, adj_vals, user_emb, item_emb, image_feats,
           text_feats, W_img, b_img, W_txt, b_txt, Wq1, bq1, Wq2, Wc1, bc1,
           Wc2):
    xn_io = _featnorm(image_feats)
    xn_to = _featnorm(text_feats)
    xn_if = _featnorm(image_feats, W_img, b_img)
    xn_tf = _featnorm(text_feats, W_txt, b_txt)

    g_io = _simtopk(xn_io)
    g_to = _simtopk(xn_to)
    g_if = _simtopk(xn_if)
    g_tf = _simtopk(xn_tf)

    image_item = _knn_prop(g_if, g_io, item_emb)
    text_item = _knn_prop(g_tf, g_to, item_emb)

    h = _attn_fuse(image_item, text_item, Wq1, bq1.reshape(1, _D), Wq2)

    ego0 = jnp.concatenate([user_emb, item_emb], axis=0)
    rows2 = adj_rows.reshape(_NNZ // _CH, _CH)
    cols2 = adj_cols.reshape(_NNZ // _CH, _CH)
    vals2 = adj_vals.reshape(_NNZ // _CH, _CH)
    p1 = _segsum_layer(rows2, cols2, vals2, ego0)
    ego1 = _pairsum(p1)
    p2 = _segsum_layer(rows2, cols2, vals2, ego1)
    allemb = _mean3(ego0, ego1, p2)
    u_g, i_g = allemb[:_N_USERS], allemb[_N_USERS:]

    i_g = _attn_fuse(i_g, h, Wc1, bc1.reshape(1, _D), Wc2, norm_b=True)
    return (u_g, i_g, image_item, text_item, h)


# speedup vs baseline: 18.5682x; 1.0095x over previous
import functools

import jax
import jax.numpy as jnp
from jax import lax
from jax.experimental import pallas as pl
from jax.experimental.pallas import tpu as pltpu
from jax.experimental.pallas import tpu_sc as plsc

_N_USERS = 16384
_N_ITEMS = 4096
_D = 64
_TOPK = 10
_LAMBDA = 0.5
_N_NODES = _N_USERS + _N_ITEMS
_BM = 256


def _simtopk_body(x_ref, xt_ref, vals_ref, idx_ref, dinv_ref):
    x = x_ref[...]
    xt = xt_ref[...]
    s = lax.dot_general(x, xt, (((1,), (1,)), ((), ())),
                        preferred_element_type=jnp.float32)
    iota = lax.broadcasted_iota(jnp.int32, (_BM, _N_ITEMS), 1).astype(
        jnp.float32)
    neg = jnp.float32(-jnp.inf)
    vs, js = [], []
    m = jnp.max(s, axis=1, keepdims=True)
    for k in range(_TOPK):
        j = jnp.sum(jnp.where(s == m, iota, 0.0), axis=1, keepdims=True)
        vs.append(m)
        js.append(j.astype(jnp.int32))
        if k + 1 < _TOPK:
            m = jnp.max(jnp.where(s < m, s, neg), axis=1, keepdims=True)
    rowsum = vs[0]
    for v in vs[1:]:
        rowsum = rowsum + v
    dinv = jnp.where(rowsum > 0, lax.rsqrt(rowsum + 1e-8), 0.0)
    pad_f = jnp.zeros((_BM, 128 - _TOPK), jnp.float32)
    pad_i = jnp.zeros((_BM, 128 - _TOPK), jnp.int32)
    vals_ref[...] = jnp.concatenate(vs + [pad_f], axis=1)
    idx_ref[...] = jnp.concatenate(js + [pad_i], axis=1)
    dinv_ref[...] = jnp.broadcast_to(dinv, (_BM, 128))


def _simtopk(xn):
    f = xn.shape[1]
    vals, idx, dinv = pl.pallas_call(
        _simtopk_body,
        grid=(_N_ITEMS // _BM,),
        in_specs=[
            pl.BlockSpec((_BM, f), lambda i: (i, 0)),
            pl.BlockSpec((_N_ITEMS, f), lambda i: (0, 0)),
        ],
        out_specs=[
            pl.BlockSpec((_BM, 128), lambda i: (i, 0)),
            pl.BlockSpec((_BM, 128), lambda i: (i, 0)),
            pl.BlockSpec((_BM, 128), lambda i: (i, 0)),
        ],
        out_shape=[
            jax.ShapeDtypeStruct((_N_ITEMS, 128), jnp.float32),
            jax.ShapeDtypeStruct((_N_ITEMS, 128), jnp.int32),
            jax.ShapeDtypeStruct((_N_ITEMS, 128), jnp.float32),
        ],
    )(xn, xn)
    return vals[:, :_TOPK], idx[:, :_TOPK], dinv[:, 0]


_NNZ = 1 << 20
_NTILES = 32
_EPT = _NNZ // _NTILES
_CH = 128
_SUPER = 32
_NSUP = _EPT // (_CH * _SUPER)
_RPT = _N_NODES // 16


def _segsum_body(rows_h, cols_h, vals_h, ego_h, out_h,
                 acc, colsb, rowsb, valsb, gath, zbuf, gsem, isem):
    c = lax.axis_index("c")
    s = lax.axis_index("s")
    wid = c * 16 + s
    chunk0 = wid * (_EPT // _CH)

    @pl.loop(0, 64)
    def _zero(i):
        z = jnp.zeros((16,), jnp.float32)
        for d in range(4):
            zbuf[i, pl.ds(d * 16, 16)] = z

    @pl.loop(0, _RPT // 64)
    def _zcp(t):
        pltpu.sync_copy(zbuf, acc.at[pl.ds(s * _RPT + t * 64, 64)])

    plsc.subcore_barrier()

    def _idx_copies(sc, sb, issue):
        sbase = chunk0 + sc * _SUPER
        op = pltpu.async_copy if issue else pltpu.make_async_copy
        return [
            op(rows_h.at[pl.ds(sbase, _SUPER)], rowsb.at[sb], isem),
            op(cols_h.at[pl.ds(sbase, _SUPER)], colsb.at[sb], isem),
            op(vals_h.at[pl.ds(sbase, _SUPER)], valsb.at[sb], isem),
        ]

    for d in _idx_copies(0, 0, True):
        d.wait()

    for sc in range(_NSUP):
        sb = sc % 2
        if sc + 1 < _NSUP:
            _idx_copies(sc + 1, 1 - sb, True)

        pltpu.async_copy(ego_h.at[colsb.at[sb].at[0]], gath.at[0], gsem).wait()

        @pl.loop(0, _SUPER)
        def _chunk(k):
            b = lax.rem(k, 2)

            @pl.when(k + 1 < _SUPER)
            def _start():
                pltpu.async_copy(ego_h.at[colsb.at[sb].at[k + 1]],
                                 gath.at[1 - b], gsem)

            @plsc.parallel_loop(0, _CH // 16, unroll=2)
            def _scale(e16):
                v16 = valsb[sb, k, pl.ds(e16 * 16, 16)]
                for j in range(16):
                    w = v16[j]
                    e = e16 * 16 + j
                    for d in range(4):
                        sl = pl.ds(d * 16, 16)
                        gath[b, e, sl] = gath[b, e, sl] * w

            pltpu.sync_copy(gath.at[b], acc.at[rowsb.at[sb].at[k]], add=True)

            @pl.when(k + 1 < _SUPER)
            def _drain():
                pltpu.make_async_copy(ego_h.at[colsb.at[sb].at[k + 1]],
                                      gath.at[1 - b], gsem).wait()

        if sc + 1 < _NSUP:
            for d in _idx_copies(sc + 1, 1 - sb, False):
                d.wait()

    plsc.subcore_barrier()
    pltpu.sync_copy(acc.at[pl.ds(s * _RPT, _RPT)],
                    out_h.at[c].at[pl.ds(s * _RPT, _RPT)])


def _segsum_layer(rows2, cols2, vals2, ego):
    f = pl.kernel(
        _segsum_body,
        out_type=jax.ShapeDtypeStruct((2, _N_NODES, _D), jnp.float32),
        mesh=plsc.VectorSubcoreMesh(core_axis_name="c", subcore_axis_name="s"),
        scratch_types=[
            pltpu.VMEM_SHARED((_N_NODES, _D), jnp.float32),
            pltpu.VMEM((2, _SUPER, _CH), jnp.int32),
            pltpu.VMEM((2, _SUPER, _CH), jnp.int32),
            pltpu.VMEM((2, _SUPER, _CH), jnp.float32),
            pltpu.VMEM((2, _CH, _D), jnp.float32),
            pltpu.VMEM((64, _D), jnp.float32),
            pltpu.SemaphoreType.DMA,
            pltpu.SemaphoreType.DMA,
        ],
        compiler_params=pltpu.CompilerParams(use_tc_tiling_on_sc=False),
    )
    return f(rows2, cols2, vals2, ego)


_PROWS = _N_ITEMS // _NTILES
_PEDGE = _PROWS * _TOPK


def _prop_body(vf1_h, if1_h, d1_h, vf2_h, if2_h, d2_h,
               emb_h, out_h, idxs, idxsf, valssf, dtabf, gath, outs, gsem):
    c = lax.axis_index("c")
    s = lax.axis_index("s")
    wid = c * 16 + s

    for phase in range(2):
        valsf_h = (vf1_h, vf2_h)[phase]
        idxf_h = (if1_h, if2_h)[phase]
        dinv_h = (d1_h, d2_h)[phase]
        lam = (1.0 - _LAMBDA, _LAMBDA)[phase]

        for gr in range(10):
            pltpu.sync_copy(
                idxf_h.at[pl.ds(wid * _PEDGE + gr * _CH, _CH)], idxs.at[gr])
        for gr in range(10):
            pltpu.async_copy(emb_h.at[idxs.at[gr]],
                             gath.at[pl.ds(gr * _CH, _CH)], gsem)
        pltpu.sync_copy(idxf_h.at[pl.ds(wid * _PEDGE, _PEDGE)],
                        idxsf.at[pl.ds(0, _PEDGE)])
        pltpu.sync_copy(valsf_h.at[pl.ds(wid * _PEDGE, _PEDGE)],
                        valssf.at[pl.ds(0, _PEDGE)])
        pltpu.sync_copy(dinv_h, dtabf.at[pl.ds(0, _N_ITEMS)])

        for gr in range(10):
            pltpu.make_async_copy(emb_h.at[idxs.at[gr]],
                                  gath.at[pl.ds(gr * _CH, _CH)], gsem).wait()

        @plsc.parallel_loop(0, _PROWS, unroll=2)
        def _acc(r):
            div = dtabf[pl.ds(wid * _PROWS + r, 16)]
            lam_di = lam * div[0]
            idxrow = idxsf[pl.ds(r * _TOPK, 16)]
            valsrow = valssf[pl.ds(r * _TOPK, 16)]
            a = [jnp.zeros((16,), jnp.float32) for _ in range(4)]
            for k in range(_TOPK):
                djv = dtabf[pl.ds(idxrow[k], 16)]
                w = lam_di * valsrow[k] * djv[0]
                for d in range(4):
                    a[d] = a[d] + w * gath[r * _TOPK + k, pl.ds(d * 16, 16)]
            for d in range(4):
                sl = pl.ds(d * 16, 16)
                if phase == 0:
                    outs[r, sl] = a[d]
                else:
                    outs[r, sl] = outs[r, sl] + a[d]

    pltpu.sync_copy(outs, out_h.at[pl.ds(wid * _PROWS, _PROWS)])


def _knn_prop(g1, g2, emb):
    v1, i1, d1 = g1
    v2, i2, d2 = g2
    f = pl.kernel(
        _prop_body,
        out_type=jax.ShapeDtypeStruct((_N_ITEMS, _D), jnp.float32),
        mesh=plsc.VectorSubcoreMesh(core_axis_name="c", subcore_axis_name="s"),
        scratch_types=[
            pltpu.VMEM((10, _CH), jnp.int32),
            pltpu.VMEM((_PEDGE + 16,), jnp.int32),
            pltpu.VMEM((_PEDGE + 16,), jnp.float32),
            pltpu.VMEM((_N_ITEMS + 16,), jnp.float32),
            pltpu.VMEM((_PEDGE, _D), jnp.float32),
            pltpu.VMEM((_PROWS, _D), jnp.float32),
            pltpu.SemaphoreType.DMA,
        ],
        compiler_params=pltpu.CompilerParams(use_tc_tiling_on_sc=False),
    )
    flat = lambda a: a.reshape(_N_ITEMS * _TOPK)
    return f(flat(v1), flat(i1), d1, flat(v2), flat(i2), d2, emb)


def _pairsum_body(p_ref, o_ref):
    o_ref[...] = p_ref[0] + p_ref[1]


def _pairsum(p):
    blk = 2048
    return pl.pallas_call(
        _pairsum_body,
        grid=(_N_NODES // blk,),
        in_specs=[pl.BlockSpec((2, blk, _D), lambda i: (0, i, 0))],
        out_specs=pl.BlockSpec((blk, _D), lambda i: (i, 0)),
        out_shape=jax.ShapeDtypeStruct((_N_NODES, _D), jnp.float32),
    )(p)


def _mean_body(e0_ref, e1_ref, p_ref, o_ref):
    o_ref[...] = (e0_ref[...] + e1_ref[...] + p_ref[0] + p_ref[1]) * (1.0 / 3.0)


def _mean3(ego0, ego1, p2):
    blk = 2048
    return pl.pallas_call(
        _mean_body,
        grid=(_N_NODES // blk,),
        in_specs=[
            pl.BlockSpec((blk, _D), lambda i: (i, 0)),
            pl.BlockSpec((blk, _D), lambda i: (i, 0)),
            pl.BlockSpec((2, blk, _D), lambda i: (0, i, 0)),
        ],
        out_specs=pl.BlockSpec((blk, _D), lambda i: (i, 0)),
        out_shape=jax.ShapeDtypeStruct((_N_NODES, _D), jnp.float32),
    )(ego0, ego1, p2)


def _normalize(x, eps):
    return x / (jnp.linalg.norm(x, axis=1, keepdims=True) + eps)


def _featnorm_body(x_ref, w_ref, b_ref, o_ref, *, transform):
    x = x_ref[...]
    if transform:
        x = lax.dot_general(x, w_ref[...], (((1,), (0,)), ((), ())),
                            preferred_element_type=jnp.float32) + b_ref[...]
    n = jnp.sqrt(jnp.sum(x * x, axis=1, keepdims=True))
    o_ref[...] = x / (n + 1e-8)


def _featnorm(x, w=None, b=None):
    transform = w is not None
    blk = 1024
    f_in = x.shape[1]
    f_out = w.shape[1] if transform else f_in
    body = functools.partial(_featnorm_body, transform=transform)
    if not transform:
        w = jnp.zeros((8, 128), jnp.float32)
        b = jnp.zeros((1, 128), jnp.float32)
    else:
        b = b.reshape(1, f_out)
    return pl.pallas_call(
        body,
        grid=(_N_ITEMS // blk,),
        in_specs=[
            pl.BlockSpec((blk, f_in), lambda i: (i, 0)),
            pl.BlockSpec(w.shape, lambda i: (0, 0)),
            pl.BlockSpec(b.shape, lambda i: (0, 0)),
        ],
        out_specs=pl.BlockSpec((blk, f_out), lambda i: (i, 0)),
        out_shape=jax.ShapeDtypeStruct((_N_ITEMS, f_out), jnp.float32),
    )(x, w, b)


def _attn_body(a_ref, b_ref, w1_ref, b1_ref, w2_ref, o_ref, *, norm_b):
    a = a_ref[...]
    bb = b_ref[...]
    if norm_b:
        n = jnp.sqrt(jnp.sum(bb * bb, axis=1, keepdims=True))
        bb = bb / (n + 1e-12)
    w1 = w1_ref[...]
    b1 = b1_ref[...]
    w2 = w2_ref[...]
    q = lambda x: lax.dot_general(
        jnp.tanh(lax.dot_general(x, w1, (((1,), (0,)), ((), ())),
                                 preferred_element_type=jnp.float32) + b1),
        w2, (((1,), (0,)), ((), ())), preferred_element_type=jnp.float32)
    qa = q(a)[:, 0:1]
    qb = q(bb)[:, 0:1]
    wa = 1.0 / (1.0 + jnp.exp(qb - qa))
    o_ref[...] = wa * a + (1.0 - wa) * bb


def _attn_fuse(a, b, w1, b1, w2, norm_b=False):
    body = functools.partial(_attn_body, norm_b=norm_b)
    w2p = jnp.pad(w2, ((0, 0), (0, 127)))
    return pl.pallas_call(
        body,
        grid=(2,),
        in_specs=[
            pl.BlockSpec((_N_ITEMS // 2, _D), lambda i: (i, 0)),
            pl.BlockSpec((_N_ITEMS // 2, _D), lambda i: (i, 0)),
            pl.BlockSpec((_D, _D), lambda i: (0, 0)),
            pl.BlockSpec((1, _D), lambda i: (0, 0)),
            pl.BlockSpec((_D, 128), lambda i: (0, 0)),
        ],
        out_specs=pl.BlockSpec((_N_ITEMS // 2, _D), lambda i: (i, 0)),
        out_shape=jax.ShapeDtypeStruct((_N_ITEMS, _D), jnp.float32),
    )(a, b, w1, b1, w2p)


def kernel(adj_rows, adj_cols, adj_vals, user_emb, item_emb, image_feats,
           text_feats, W_img, b_img, W_txt, b_txt, Wq1, bq1, Wq2, Wc1, bc1,
           Wc2):
    xn_io = _featnorm(image_feats)
    xn_to = _featnorm(text_feats)
    xn_if = _featnorm(image_feats, W_img, b_img)
    xn_tf = _featnorm(text_feats, W_txt, b_txt)

    g_io = _simtopk(xn_io)
    g_to = _simtopk(xn_to)
    g_if = _simtopk(xn_if)
    g_tf = _simtopk(xn_tf)

    image_item = _knn_prop(g_if, g_io, item_emb)
    text_item = _knn_prop(g_tf, g_to, item_emb)

    h = _attn_fuse(image_item, text_item, Wq1, bq1.reshape(1, _D), Wq2)

    ego0 = jnp.concatenate([user_emb, item_emb], axis=0)
    rows2 = adj_rows.reshape(_NNZ // _CH, _CH)
    cols2 = adj_cols.reshape(_NNZ // _CH, _CH)
    vals2 = adj_vals.reshape(_NNZ // _CH, _CH)
    p1 = _segsum_layer(rows2, cols2, vals2, ego0)
    ego1 = _pairsum(p1)
    p2 = _segsum_layer(rows2, cols2, vals2, ego1)
    allemb = _mean3(ego0, ego1, p2)
    u_g, i_g = allemb[:_N_USERS], allemb[_N_USERS:]

    i_g = _attn_fuse(i_g, h, Wc1, bc1.reshape(1, _D), Wc2, norm_b=True)
    return (u_g, i_g, image_item, text_item, h)
